# baseline (speedup 1.0000x reference)
import functools

import jax
import jax.numpy as jnp
from jax import lax
from jax.experimental import pallas as pl
from jax.experimental.pallas import tpu as pltpu
from jax.experimental.pallas import tpu_sc as plsc

H = 16
D = 1024
F = 4096
N = 8192
T = 256
NPT = 12288
G = 48
NB = NPT // T


@functools.cache
def _make_row_gather(n_idx, chunk):
    nw = 32
    per_w = n_idx // nw
    n_ch = per_w // chunk
    assert per_w % chunk == 0 and chunk % 8 == 0 and chunk <= 128
    assert 2 * chunk * D + per_w < 131071
    mesh = plsc.VectorSubcoreMesh(core_axis_name="c", subcore_axis_name="s")

    @functools.partial(
        pl.kernel,
        mesh=mesh,
        out_type=jax.ShapeDtypeStruct((n_idx, D), jnp.float32),
        scratch_types=[
            pltpu.VMEM((per_w,), jnp.int32),
            pltpu.VMEM((2, chunk, D), jnp.float32),
            pltpu.SemaphoreType.DMA,
            pltpu.SemaphoreType.DMA,
        ],
    )
    def gather_k(table_hbm, idx_hbm, out_hbm, idx_v, rows_v, sem_a, sem_b):
        wid = lax.axis_index("s") * 2 + lax.axis_index("c")
        base = wid * per_w
        pltpu.sync_copy(idx_hbm.at[pl.ds(base, per_w)], idx_v)
        sems = (sem_a, sem_b)

        def start(i):
            pltpu.async_copy(
                table_hbm.at[idx_v.at[pl.ds(i * chunk, chunk)]],
                rows_v.at[i % 2], sems[i % 2])

        start(0)
        for i in range(n_ch):
            if i + 1 < n_ch:
                start(i + 1)
            pltpu.make_async_copy(
                table_hbm.at[idx_v.at[pl.ds(i * chunk, chunk)]],
                rows_v.at[i % 2], sems[i % 2]).wait()
            pltpu.sync_copy(rows_v.at[i % 2],
                            out_hbm.at[pl.ds(base + i * chunk, chunk)])

    return gather_k


def _layer1_body(eids_ref, bidx_ref, valid_ref, x_ref, w1_ref, b1_ref, h_ref):
    g = pl.program_id(0)

    @pl.when(valid_ref[g] != 0)
    def _():
        h_ref[...] = jnp.maximum(
            jnp.dot(x_ref[...].astype(jnp.bfloat16),
                    w1_ref[0].astype(jnp.bfloat16),
                    preferred_element_type=jnp.float32)
            + b1_ref[0], 0.0).astype(jnp.bfloat16)


_layer1_call = pl.pallas_call(
    _layer1_body,
    grid_spec=pltpu.PrefetchScalarGridSpec(
        num_scalar_prefetch=3,
        grid=(G,),
        in_specs=[
            pl.BlockSpec((T, D), lambda g, eids, bidx, valid: (bidx[g], 0)),
            pl.BlockSpec((1, D, F), lambda g, eids, bidx, valid: (eids[g], 0, 0)),
            pl.BlockSpec((1, 1, F), lambda g, eids, bidx, valid: (eids[g], 0, 0)),
        ],
        out_specs=pl.BlockSpec((T, F), lambda g, eids, bidx, valid: (bidx[g], 0)),
    ),
    out_shape=jax.ShapeDtypeStruct((NPT, F), jnp.bfloat16),
)


def _layer2_body(eids_ref, bidx_ref, valid_ref, h_ref, w2_ref, b2_ref, y_ref):
    g = pl.program_id(0)

    @pl.when(valid_ref[g] != 0)
    def _():
        y_ref[...] = (
            jnp.dot(h_ref[...],
                    w2_ref[0].astype(jnp.bfloat16),
                    preferred_element_type=jnp.float32)
            + b2_ref[0])


_layer2_call = pl.pallas_call(
    _layer2_body,
    grid_spec=pltpu.PrefetchScalarGridSpec(
        num_scalar_prefetch=3,
        grid=(G,),
        in_specs=[
            pl.BlockSpec((T, F), lambda g, eids, bidx, valid: (bidx[g], 0)),
            pl.BlockSpec((1, F, D), lambda g, eids, bidx, valid: (eids[g], 0, 0)),
            pl.BlockSpec((1, 1, D), lambda g, eids, bidx, valid: (eids[g], 0, 0)),
        ],
        out_specs=pl.BlockSpec((T, D), lambda g, eids, bidx, valid: (bidx[g], 0)),
    ),
    out_shape=jax.ShapeDtypeStruct((NPT, D), jnp.float32),
)


def kernel(x, orig_input, W1, b1, W2, b2, hash_bin_map):
    Bq, Sq, _ = x.shape
    xf = x.reshape(N, D)
    ids = orig_input.reshape(-1).astype(jnp.int32)
    buckets = jnp.take(hash_bin_map.astype(jnp.int32), ids)

    counts = jnp.bincount(buckets, length=H).astype(jnp.int32)
    apad = ((counts + T - 1) // T) * T
    astart = (jnp.cumsum(apad) - apad).astype(jnp.int32)
    cstart = (jnp.cumsum(counts) - counts).astype(jnp.int32)
    order = jnp.argsort(buckets, stable=True).astype(jnp.int32)
    sb = buckets[order]
    pj = jnp.arange(N, dtype=jnp.int32) + (astart - cstart)[sb]
    gather_idx = (jnp.arange(NPT, dtype=jnp.int32) % N).at[pj].set(order)
    inv_idx = jnp.zeros(N, jnp.int32).at[order].set(pj)

    tiles = apad // T
    ctiles = jnp.cumsum(tiles).astype(jnp.int32)
    total = ctiles[H - 1]
    gi = jnp.arange(G, dtype=jnp.int32)
    hg = jnp.searchsorted(ctiles, gi, side="right").astype(jnp.int32)
    hgc = jnp.minimum(hg, H - 1)
    prev = jnp.where(hg > 0, ctiles[jnp.maximum(hg, 1) - 1], 0)
    bidx0 = astart[hgc] // T + (gi - prev)
    valid = (gi < total).astype(jnp.int32)
    e_last = jnp.searchsorted(ctiles, total - 1, side="right").astype(jnp.int32)
    expert_ids = jnp.where(valid == 1, hgc, e_last)
    bidx = jnp.where(valid == 1, bidx0, NB - 1)

    xs = _make_row_gather(NPT, 48)(xf, gather_idx)
    hbuf = _layer1_call(expert_ids, bidx, valid, xs, W1, b1.reshape(H, 1, F))
    ys = _layer2_call(expert_ids, bidx, valid, hbuf, W2, b2.reshape(H, 1, D))
    out = _make_row_gather(N, 32)(ys, inv_idx)
    return out.reshape(Bq, Sq, D)


# speedup vs baseline: 1.0083x; 1.0083x over previous
import functools

import jax
import jax.numpy as jnp
from jax import lax
from jax.experimental import pallas as pl
from jax.experimental.pallas import tpu as pltpu
from jax.experimental.pallas import tpu_sc as plsc

H = 16
D = 1024
F = 4096
N = 8192
T = 256
NPT = 12288
G = 48
NB = NPT // T


@functools.cache
def _make_row_gather(n_idx, chunk):
    nw = 32
    per_w = n_idx // nw
    n_ch = per_w // chunk
    assert per_w % chunk == 0 and chunk % 8 == 0 and chunk <= 128
    mesh = plsc.VectorSubcoreMesh(core_axis_name="c", subcore_axis_name="s")

    @functools.partial(
        pl.kernel,
        mesh=mesh,
        out_type=jax.ShapeDtypeStruct((n_idx, D), jnp.float32),
        scratch_types=[
            pltpu.VMEM((chunk,), jnp.int32),
            pltpu.VMEM((chunk, D), jnp.float32),
            pltpu.SemaphoreType.DMA,
        ],
    )
    def gather_k(table_hbm, idx_hbm, out_hbm, idx_v, rows_v, sem):
        wid = lax.axis_index("s") * 2 + lax.axis_index("c")
        base = wid * per_w

        def body(i, carry):
            off = base + i * chunk
            pltpu.sync_copy(idx_hbm.at[pl.ds(off, chunk)], idx_v)
            pltpu.async_copy(table_hbm.at[idx_v], rows_v, sem).wait()
            pltpu.sync_copy(rows_v, out_hbm.at[pl.ds(off, chunk)])
            return carry

        lax.fori_loop(0, n_ch, body, 0)

    return gather_k


def _layer1_body(eids_ref, bidx_ref, valid_ref, x_ref, w1_ref, b1_ref, h_ref):
    g = pl.program_id(0)

    @pl.when(valid_ref[g] != 0)
    def _():
        h_ref[...] = jnp.maximum(
            jnp.dot(x_ref[...].astype(jnp.bfloat16),
                    w1_ref[0].astype(jnp.bfloat16),
                    preferred_element_type=jnp.float32)
            + b1_ref[0], 0.0).astype(jnp.bfloat16)


_layer1_call = pl.pallas_call(
    _layer1_body,
    grid_spec=pltpu.PrefetchScalarGridSpec(
        num_scalar_prefetch=3,
        grid=(G,),
        in_specs=[
            pl.BlockSpec((T, D), lambda g, eids, bidx, valid: (bidx[g], 0)),
            pl.BlockSpec((1, D, F), lambda g, eids, bidx, valid: (eids[g], 0, 0)),
            pl.BlockSpec((1, 1, F), lambda g, eids, bidx, valid: (eids[g], 0, 0)),
        ],
        out_specs=pl.BlockSpec((T, F), lambda g, eids, bidx, valid: (bidx[g], 0)),
    ),
    out_shape=jax.ShapeDtypeStruct((NPT, F), jnp.bfloat16),
)


def _layer2_body(eids_ref, bidx_ref, valid_ref, h_ref, w2_ref, b2_ref, y_ref):
    g = pl.program_id(0)

    @pl.when(valid_ref[g] != 0)
    def _():
        y_ref[...] = (
            jnp.dot(h_ref[...],
                    w2_ref[0].astype(jnp.bfloat16),
                    preferred_element_type=jnp.float32)
            + b2_ref[0])


_layer2_call = pl.pallas_call(
    _layer2_body,
    grid_spec=pltpu.PrefetchScalarGridSpec(
        num_scalar_prefetch=3,
        grid=(G,),
        in_specs=[
            pl.BlockSpec((T, F), lambda g, eids, bidx, valid: (bidx[g], 0)),
            pl.BlockSpec((1, F, D), lambda g, eids, bidx, valid: (eids[g], 0, 0)),
            pl.BlockSpec((1, 1, D), lambda g, eids, bidx, valid: (eids[g], 0, 0)),
        ],
        out_specs=pl.BlockSpec((T, D), lambda g, eids, bidx, valid: (bidx[g], 0)),
    ),
    out_shape=jax.ShapeDtypeStruct((NPT, D), jnp.float32),
)


def kernel(x, orig_input, W1, b1, W2, b2, hash_bin_map):
    Bq, Sq, _ = x.shape
    xf = x.reshape(N, D)
    ids = orig_input.reshape(-1).astype(jnp.int32)
    buckets = jnp.take(hash_bin_map.astype(jnp.int32), ids)

    counts = jnp.bincount(buckets, length=H).astype(jnp.int32)
    apad = ((counts + T - 1) // T) * T
    astart = (jnp.cumsum(apad) - apad).astype(jnp.int32)
    cstart = (jnp.cumsum(counts) - counts).astype(jnp.int32)
    order = jnp.argsort(buckets, stable=True).astype(jnp.int32)
    sb = buckets[order]
    pj = jnp.arange(N, dtype=jnp.int32) + (astart - cstart)[sb]
    gather_idx = (jnp.arange(NPT, dtype=jnp.int32) % N).at[pj].set(order)
    inv_idx = jnp.zeros(N, jnp.int32).at[order].set(pj)

    tiles = apad // T
    ctiles = jnp.cumsum(tiles).astype(jnp.int32)
    total = ctiles[H - 1]
    gi = jnp.arange(G, dtype=jnp.int32)
    hg = jnp.searchsorted(ctiles, gi, side="right").astype(jnp.int32)
    hgc = jnp.minimum(hg, H - 1)
    prev = jnp.where(hg > 0, ctiles[jnp.maximum(hg, 1) - 1], 0)
    bidx0 = astart[hgc] // T + (gi - prev)
    valid = (gi < total).astype(jnp.int32)
    e_last = jnp.searchsorted(ctiles, total - 1, side="right").astype(jnp.int32)
    expert_ids = jnp.where(valid == 1, hgc, e_last)
    bidx = jnp.where(valid == 1, bidx0, NB - 1)

    xs = _make_row_gather(NPT, 96)(xf, gather_idx)
    hbuf = _layer1_call(expert_ids, bidx, valid, xs, W1, b1.reshape(H, 1, F))
    ys = _layer2_call(expert_ids, bidx, valid, hbuf, W2, b2.reshape(H, 1, D))
    out = _make_row_gather(N, 64)(ys, inv_idx)
    return out.reshape(Bq, Sq, D)
